# Initial kernel scaffold; baseline (speedup 1.0000x reference)
import jax
import jax.numpy as jnp
from jax.experimental import pallas as pl


def kernel(hidden_states, gate_w, w1, w2, w3):
    raise NotImplementedError("write your pallas kernel here")


# speedup vs baseline: 1.4362x; 1.4362x over previous
import functools
import jax
import jax.numpy as jnp
from jax import lax
from jax.experimental import pallas as pl
from jax.experimental.pallas import tpu as pltpu

HIDDEN = 1024
FFN = 3584
NUM_EXPERTS = 8
TOP_K = 2

TB = 1024
FB = 896
NF = FFN // FB


def _moe_body(x_ref, gate_ref, w1_ref, w3_ref, w2_ref, out_ref, dw_ref):
    e = pl.program_id(1)
    f = pl.program_id(2)

    @pl.when((e == 0) & (f == 0))
    def _router():
        x = x_ref[...]
        logits = lax.dot_general(
            x, gate_ref[...], (((1,), (1,)), ((), ())),
            preferred_element_type=jnp.float32)
        m = jnp.max(logits, axis=-1, keepdims=True)
        p = jnp.exp(logits - m)
        rw = p / jnp.sum(p, axis=-1, keepdims=True)
        lane = lax.broadcasted_iota(jnp.int32, rw.shape, 1)
        m1 = jnp.max(rw, axis=-1, keepdims=True)
        i1 = jnp.min(jnp.where(rw == m1, lane, NUM_EXPERTS), axis=-1,
                     keepdims=True)
        rw2 = jnp.where(lane == i1, -jnp.inf, rw)
        m2 = jnp.max(rw2, axis=-1, keepdims=True)
        i2 = jnp.min(jnp.where(rw2 == m2, lane, NUM_EXPERTS), axis=-1,
                     keepdims=True)
        s = m1 + m2
        dw = (jnp.where(lane == i1, m1 / s, 0.0)
              + jnp.where(lane == i2, m2 / s, 0.0))
        dw_ref[...] = dw
        out_ref[...] = jnp.zeros_like(out_ref)

    x = x_ref[...]
    h1 = lax.dot_general(x, w1_ref[0], (((1,), (1,)), ((), ())),
                         preferred_element_type=jnp.float32)
    h3 = lax.dot_general(x, w3_ref[0], (((1,), (1,)), ((), ())),
                         preferred_element_type=jnp.float32)
    act = h1 * (1.0 / (1.0 + jnp.exp(-h1))) * h3
    y = lax.dot_general(act, w2_ref[0], (((1,), (1,)), ((), ())),
                        preferred_element_type=jnp.float32)
    lane = lax.broadcasted_iota(jnp.int32, (TB, NUM_EXPERTS), 1)
    wcol = jnp.sum(jnp.where(lane == e, dw_ref[...], 0.0), axis=-1,
                   keepdims=True)
    out_ref[...] += wcol * y


@jax.jit
def _moe(x, gate_w, w1, w2, w3):
    T = x.shape[0]
    grid = (T // TB, NUM_EXPERTS, NF)
    return pl.pallas_call(
        _moe_body,
        grid=grid,
        in_specs=[
            pl.BlockSpec((TB, HIDDEN), lambda t, e, f: (t, 0)),
            pl.BlockSpec((NUM_EXPERTS, HIDDEN), lambda t, e, f: (0, 0)),
            pl.BlockSpec((1, FB, HIDDEN), lambda t, e, f: (e, f, 0)),
            pl.BlockSpec((1, FB, HIDDEN), lambda t, e, f: (e, f, 0)),
            pl.BlockSpec((1, HIDDEN, FB), lambda t, e, f: (e, 0, f)),
        ],
        out_specs=pl.BlockSpec((TB, HIDDEN), lambda t, e, f: (t, 0)),
        out_shape=jax.ShapeDtypeStruct((T, HIDDEN), jnp.float32),
        scratch_shapes=[pltpu.VMEM((TB, NUM_EXPERTS), jnp.float32)],
        compiler_params=pltpu.CompilerParams(
            dimension_semantics=("parallel", "arbitrary", "arbitrary"),
        ),
    )(x, gate_w, w1, w3, w2)


def kernel(hidden_states, gate_w, w1, w2, w3):
    B, S, H = hidden_states.shape
    x = hidden_states.reshape(-1, H)
    out = _moe(x, gate_w, w1, w2, w3)
    return out.reshape(B, S, H)
